# baseline (speedup 1.0000x reference)
import jax
import jax.numpy as jnp
from jax.experimental import pallas as pl
from jax.experimental.pallas import tpu as pltpu

_K = 8
_S = 512
_STRIDE = 9


def _diag_weight() -> jnp.ndarray:
    u = jnp.arange(_S, dtype=jnp.int32)[:, None]
    v = jnp.arange(_S, dtype=jnp.int32)[None, :]
    delta = v - u
    plus = ((delta % _STRIDE == 0) & (delta >= 0)).astype(jnp.float32)
    minus = (((delta - 1) % _STRIDE == 0) & (delta - 1 >= 0)).astype(jnp.float32)
    return (_K * (plus - minus)).astype(jnp.bfloat16)


def _collect_weight(nb: int) -> jnp.ndarray:
    u = jnp.arange(_S, dtype=jnp.int32)[None, :, None]
    r = jnp.arange(16, dtype=jnp.int32)[None, None, :]
    j = jnp.arange(nb, dtype=jnp.int32)[:, None, None]
    p = ((u % _STRIDE == (r + j) % _STRIDE) & (r < _STRIDE)).astype(jnp.float32)
    p = p.at[0, 0, _STRIDE].set(1.0)
    return p.astype(jnp.bfloat16)


def _expand_weight(nb: int) -> jnp.ndarray:
    v = jnp.arange(_S, dtype=jnp.int32)[None, None, :]
    r = jnp.arange(16, dtype=jnp.int32)[None, :, None]
    i = jnp.arange(nb, dtype=jnp.int32)[:, None, None]
    f = ((v - i) % _STRIDE == r).astype(jnp.float32) - \
        ((v - i - 1) % _STRIDE == r).astype(jnp.float32)
    edge = ((v - i) % _STRIDE == 5).astype(jnp.float32) - \
           ((v - i) % _STRIDE == 0).astype(jnp.float32)
    f = jnp.where(r == _STRIDE, jnp.broadcast_to(edge, f.shape), f)
    f = jnp.where(r > _STRIDE, 0.0, f)
    return _K * f


def _edge_row() -> jnp.ndarray:
    t = jnp.arange(_S, dtype=jnp.int32)[None, :]
    return _K * ((t % _STRIDE == 5).astype(jnp.float32)
                 - (t % _STRIDE == 0).astype(jnp.float32))


def _comb_kernel(x_ref, w0_ref, p_ref, f_ref, c_ref, out_ref, u_ref):
    i = pl.program_id(1)
    dn = (((1,), (0,)), ((), ()))

    @pl.when(i == 0)
    def _():
        u_ref[...] = jnp.zeros_like(u_ref)

    xb = x_ref[...].astype(jnp.bfloat16)
    acc = jax.lax.dot_general(xb, w0_ref[...], dn,
                              preferred_element_type=jnp.float32)
    carry = jax.lax.dot_general(u_ref[...], f_ref[0], dn,
                                preferred_element_type=jnp.float32)
    edge = jnp.where(i == 0, x_ref[:, 0:1] * c_ref[...], 0.0)
    out_ref[...] = acc + carry + edge
    u_ref[...] += jax.lax.dot_general(xb, p_ref[0], dn,
                                      preferred_element_type=jnp.float32)


@jax.jit
def kernel(x) -> jnp.ndarray:
    B, C, T = x.shape
    nb = T // _S
    rows = B * C
    R = 2048 if rows % 2048 == 0 else rows
    x2 = x.reshape(rows, T)

    out = pl.pallas_call(
        _comb_kernel,
        grid=(rows // R, nb),
        in_specs=[
            pl.BlockSpec((R, _S), lambda o, i: (o, i)),
            pl.BlockSpec((_S, _S), lambda o, i: (0, 0)),
            pl.BlockSpec((1, _S, 16), lambda o, i: (i, 0, 0)),
            pl.BlockSpec((1, 16, _S), lambda o, i: (i, 0, 0)),
            pl.BlockSpec((1, _S), lambda o, i: (0, 0)),
        ],
        out_specs=pl.BlockSpec((R, _S), lambda o, i: (o, i)),
        out_shape=jax.ShapeDtypeStruct((rows, T), jnp.float32),
        scratch_shapes=[pltpu.VMEM((R, 16), jnp.float32)],
        compiler_params=pltpu.CompilerParams(
            dimension_semantics=("arbitrary", "arbitrary"),
        ),
    )(x2, _diag_weight(), _collect_weight(nb), _expand_weight(nb), _edge_row())
    return out.reshape(B, C, T)


# speedup vs baseline: 1.1031x; 1.1031x over previous
import functools

import jax
import jax.numpy as jnp
from jax.experimental import pallas as pl
from jax.experimental.pallas import tpu as pltpu

_K = 8
_S = 512
_STRIDE = 9


def _diag_weight() -> jnp.ndarray:
    u = jnp.arange(_S, dtype=jnp.int32)[:, None]
    v = jnp.arange(_S, dtype=jnp.int32)[None, :]
    delta = v - u
    plus = ((delta % _STRIDE == 0) & (delta >= 0)).astype(jnp.float32)
    minus = (((delta - 1) % _STRIDE == 0) & (delta - 1 >= 0)).astype(jnp.float32)
    return (_K * (plus - minus)).astype(jnp.bfloat16)


def _collect_weight(nb: int) -> jnp.ndarray:
    u = jnp.arange(_S, dtype=jnp.int32)[None, :, None]
    r = jnp.arange(16, dtype=jnp.int32)[None, None, :]
    j = jnp.arange(nb, dtype=jnp.int32)[:, None, None]
    p = (u % _STRIDE == (r + j) % _STRIDE) & (r < _STRIDE)
    return p.astype(jnp.bfloat16)


def _expand_weight(nb: int) -> jnp.ndarray:
    v = jnp.arange(_S, dtype=jnp.int32)[None, None, :]
    r = jnp.arange(16, dtype=jnp.int32)[None, :, None]
    i = jnp.arange(nb, dtype=jnp.int32)[:, None, None]
    f = ((v - i) % _STRIDE == r).astype(jnp.float32) - \
        ((v - i - 1) % _STRIDE == r).astype(jnp.float32)
    return _K * f


def _edge_row(T: int) -> jnp.ndarray:
    t = jnp.arange(T, dtype=jnp.int32)[None, :]
    return _K * ((t % _STRIDE == 5).astype(jnp.float32)
                 - (t % _STRIDE == 0).astype(jnp.float32))


def _comb_kernel(x_ref, w0_ref, p_ref, f_ref, c_ref, out_ref, *, nb: int):
    x0 = x_ref[:, 0:1]
    dn = (((1,), (0,)), ((), ()))
    u = None
    for i in range(nb):
        xb = x_ref[:, i * _S:(i + 1) * _S].astype(jnp.bfloat16)
        acc = jax.lax.dot_general(xb, w0_ref[...], dn,
                                  preferred_element_type=jnp.float32)
        acc += x0 * c_ref[:, i * _S:(i + 1) * _S]
        if u is not None:
            acc += jax.lax.dot_general(u, f_ref[i], dn,
                                       preferred_element_type=jnp.float32)
        out_ref[:, i * _S:(i + 1) * _S] = acc
        if i + 1 < nb:
            s = jax.lax.dot_general(xb, p_ref[i], dn,
                                    preferred_element_type=jnp.float32)
            u = s if u is None else u + s


@jax.jit
def kernel(x) -> jnp.ndarray:
    B, C, T = x.shape
    nb = T // _S
    rows = B * C
    R = 512 if rows % 512 == 0 else rows
    x2 = x.reshape(rows, T)

    out = pl.pallas_call(
        functools.partial(_comb_kernel, nb=nb),
        grid=(rows // R,),
        in_specs=[
            pl.BlockSpec((R, T), lambda i: (i, 0)),
            pl.BlockSpec((_S, _S), lambda i: (0, 0)),
            pl.BlockSpec((nb, _S, 16), lambda i: (0, 0, 0)),
            pl.BlockSpec((nb, 16, _S), lambda i: (0, 0, 0)),
            pl.BlockSpec((1, T), lambda i: (0, 0)),
        ],
        out_specs=pl.BlockSpec((R, T), lambda i: (i, 0)),
        out_shape=jax.ShapeDtypeStruct((rows, T), jnp.float32),
        compiler_params=pltpu.CompilerParams(
            dimension_semantics=("arbitrary",),
        ),
    )(x2, _diag_weight(), _collect_weight(nb), _expand_weight(nb), _edge_row(T))
    return out.reshape(B, C, T)


# speedup vs baseline: 1.1594x; 1.0510x over previous
import functools

import jax
import jax.numpy as jnp
from jax.experimental import pallas as pl
from jax.experimental.pallas import tpu as pltpu

_K = 8
_S = 256
_SIG = _S % 9
_STRIDE = 9


def _diag_weight() -> jnp.ndarray:
    u = jnp.arange(_S, dtype=jnp.int32)[:, None]
    v = jnp.arange(_S, dtype=jnp.int32)[None, :]
    delta = v - u
    plus = ((delta % _STRIDE == 0) & (delta >= 0)).astype(jnp.float32)
    minus = (((delta - 1) % _STRIDE == 0) & (delta - 1 >= 0)).astype(jnp.float32)
    return (_K * (plus - minus)).astype(jnp.bfloat16)


def _collect_weight(nb: int) -> jnp.ndarray:
    u = jnp.arange(_S, dtype=jnp.int32)[None, :, None]
    r = jnp.arange(16, dtype=jnp.int32)[None, None, :]
    j = jnp.arange(nb, dtype=jnp.int32)[:, None, None]
    p = (u % _STRIDE == (r - _SIG * j) % _STRIDE) & (r < _STRIDE)
    return p.astype(jnp.bfloat16)


def _expand_weight(nb: int) -> jnp.ndarray:
    v = jnp.arange(_S, dtype=jnp.int32)[None, None, :]
    r = jnp.arange(16, dtype=jnp.int32)[None, :, None]
    i = jnp.arange(nb, dtype=jnp.int32)[:, None, None]
    f = ((v + _SIG * i) % _STRIDE == r).astype(jnp.float32) - \
        ((v - 1 + _SIG * i) % _STRIDE == r).astype(jnp.float32)
    return _K * f


def _edge_row(T: int) -> jnp.ndarray:
    t = jnp.arange(T, dtype=jnp.int32)[None, :]
    return _K * ((t % _STRIDE == 5).astype(jnp.float32)
                 - (t % _STRIDE == 0).astype(jnp.float32))


def _comb_kernel(x_ref, w0_ref, p_ref, f_ref, c_ref, out_ref, *, nb: int):
    x0 = x_ref[:, 0:1]
    dn = (((1,), (0,)), ((), ()))
    u = None
    for i in range(nb):
        xb = x_ref[:, i * _S:(i + 1) * _S].astype(jnp.bfloat16)
        acc = jax.lax.dot_general(xb, w0_ref[...], dn,
                                  preferred_element_type=jnp.float32)
        acc += x0 * c_ref[:, i * _S:(i + 1) * _S]
        if u is not None:
            acc += jax.lax.dot_general(u, f_ref[i], dn,
                                       preferred_element_type=jnp.float32)
        out_ref[:, i * _S:(i + 1) * _S] = acc
        if i + 1 < nb:
            s = jax.lax.dot_general(xb, p_ref[i], dn,
                                    preferred_element_type=jnp.float32)
            u = s if u is None else u + s


@jax.jit
def kernel(x) -> jnp.ndarray:
    B, C, T = x.shape
    nb = T // _S
    rows = B * C
    R = 512 if rows % 512 == 0 else rows
    x2 = x.reshape(rows, T)

    out = pl.pallas_call(
        functools.partial(_comb_kernel, nb=nb),
        grid=(rows // R,),
        in_specs=[
            pl.BlockSpec((R, T), lambda i: (i, 0)),
            pl.BlockSpec((_S, _S), lambda i: (0, 0)),
            pl.BlockSpec((nb, _S, 16), lambda i: (0, 0, 0)),
            pl.BlockSpec((nb, 16, _S), lambda i: (0, 0, 0)),
            pl.BlockSpec((1, T), lambda i: (0, 0)),
        ],
        out_specs=pl.BlockSpec((R, T), lambda i: (i, 0)),
        out_shape=jax.ShapeDtypeStruct((rows, T), jnp.float32),
        compiler_params=pltpu.CompilerParams(
            dimension_semantics=("arbitrary",),
        ),
    )(x2, _diag_weight(), _collect_weight(nb), _expand_weight(nb), _edge_row(T))
    return out.reshape(B, C, T)
